# baseline (speedup 1.0000x reference)
import functools

import jax
import jax.numpy as jnp
import numpy as np
from jax import lax
from jax.experimental import pallas as pl
from jax.experimental.pallas import tpu as pltpu
from jax.experimental.pallas import tpu_sc as plsc


def _pairs():
    arg1s = [[8, 9], [17, 18], [26, 27]]
    arg2s = [[11, 12, 13, 14, 15, 16], [20, 21, 22, 23, 24, 25],
             [29, 30, 31, 32, 33, 34]]
    prods = []
    for a, b in zip(arg1s, arg2s):
        for i in a:
            for j in b:
                prods.append((i, j))
    return np.array(prods, dtype=np.int32)


_P = _pairs()
_D0, _D1, _D2 = 2048, 512, 64
_NC, _NS = 2, 16
_NW = _NC * _NS
_B0 = 8
_BL = 128
_NU = (_D0 // _B0) * (_D1 // _BL)
_UPW = _NU // _NW
_LB = _D1 // _BL

_mesh = plsc.VectorSubcoreMesh(core_axis_name="c", subcore_axis_name="s")


@functools.partial(
    pl.kernel,
    out_type=jax.ShapeDtypeStruct((100, _D0, _D1), jnp.float32),
    mesh=_mesh,
    scratch_types=[
        pltpu.VMEM((_B0, 8, 8, _BL), jnp.float32),
        pltpu.VMEM((2, 8, _B0, _BL), jnp.float32),
        pltpu.VMEM((36, _B0, _BL), jnp.float32),
        pltpu.SemaphoreType.DMA,
        pltpu.SemaphoreType.DMA,
    ],
    compiler_params=pltpu.CompilerParams(use_tc_tiling_on_sc=True,
                                         needs_layout_passes=False),
)
def _sc_kernel(xq_hbm, out_hbm, in_v, cv, pv, sem0, sem1):
    wid = lax.axis_index("s") * _NC + lax.axis_index("c")
    sems = [sem0, sem1]
    srcs = sorted({int(p) for p in _P.flatten()})

    def unit_body(ui, carry):
        u = wid * _UPW + ui
        d0 = pl.multiple_of((u // _LB) * _B0, _B0)
        l0 = pl.multiple_of((u % _LB) * _BL, _BL)
        pltpu.sync_copy(
            xq_hbm.at[pl.ds(d0, _B0), :, :, pl.ds(l0, _BL)], in_v)

        pending = [None, None]
        for o in range(8):
            par = o % 2
            if pending[par] is not None:
                pending[par].wait()

            @plsc.parallel_loop(0, _B0, unroll=4)
            def tbody(d, _o=o, _par=par):
                vals = [(j, li,
                         in_v[d, _o, j, pl.ds(li * 16, 16)])
                        for j in range(8) for li in range(_BL // 16)]
                for j, li, v in vals:
                    cv[_par, j, d, pl.ds(li * 16, 16)] = v
            pending[par] = pltpu.async_copy(
                cv.at[par],
                out_hbm.at[pl.ds(8 * o, 8), pl.ds(d0, _B0),
                           pl.ds(l0, _BL)],
                sems[par])

        @plsc.parallel_loop(0, _B0, unroll=4)
        def pbody(d):
            for li in range(_BL // 16):
                l = li * 16
                src = {p: in_v[d, p // 8, p % 8, pl.ds(l, 16)]
                       for p in srcs}
                for k in range(36):
                    pv[k, d, pl.ds(l, 16)] = (
                        src[_P[k, 0]] * src[_P[k, 1]])
        for par in range(2):
            if pending[par] is not None:
                pending[par].wait()
        pltpu.sync_copy(
            pv, out_hbm.at[pl.ds(64, 36), pl.ds(d0, _B0), pl.ds(l0, _BL)])
        return carry

    lax.fori_loop(0, _UPW, unit_body, 0)


@jax.jit
def kernel(x):
    xq = jnp.transpose(x, (0, 2, 1)).reshape(_D0, 8, 8, _D1)
    ot = _sc_kernel(xq)
    return jnp.transpose(ot, (1, 2, 0))


# speedup vs baseline: 1.5306x; 1.5306x over previous
import functools

import jax
import jax.numpy as jnp
import numpy as np
from jax import lax
from jax.experimental import pallas as pl
from jax.experimental.pallas import tpu as pltpu
from jax.experimental.pallas import tpu_sc as plsc


def _pairs():
    arg1s = [[8, 9], [17, 18], [26, 27]]
    arg2s = [[11, 12, 13, 14, 15, 16], [20, 21, 22, 23, 24, 25],
             [29, 30, 31, 32, 33, 34]]
    prods = []
    for a, b in zip(arg1s, arg2s):
        for i in a:
            for j in b:
                prods.append((i, j))
    return np.array(prods, dtype=np.int32)


_P = _pairs()
_D0, _D1, _D2 = 2048, 512, 64
_NC, _NS = 2, 16
_NW = _NC * _NS
_B0 = 8
_BL = 128
_NU = (_D0 // _B0) * (_D1 // _BL)
_UPW = _NU // _NW
_LB = _D1 // _BL

_mesh = plsc.VectorSubcoreMesh(core_axis_name="c", subcore_axis_name="s")


@functools.partial(
    pl.kernel,
    out_type=jax.ShapeDtypeStruct((100, _D0, _D1), jnp.float32),
    mesh=_mesh,
    scratch_types=[
        pltpu.VMEM((_B0, 8, 8, _BL), jnp.float32),
        pltpu.VMEM((2, 8, _B0, _BL), jnp.float32),
        pltpu.VMEM((2, 18, _B0, _BL), jnp.float32),
        pltpu.SemaphoreType.DMA,
        pltpu.SemaphoreType.DMA,
        pltpu.SemaphoreType.DMA,
        pltpu.SemaphoreType.DMA,
        pltpu.SemaphoreType.DMA,
        pltpu.SemaphoreType.DMA,
    ],
    compiler_params=pltpu.CompilerParams(use_tc_tiling_on_sc=True,
                                         needs_layout_passes=False),
)
def _sc_kernel(xq_hbm, out_hbm, in_v, cv, pv,
               sem0, sem1, semi0, semi1, semp0, semp1):
    wid = lax.axis_index("s") * _NC + lax.axis_index("c")
    sems = [sem0, sem1]
    semi = [semi0, semi1]
    semp = [semp0, semp1]
    srcs = sorted({int(p) for p in _P.flatten()})

    def unit_body(ui, carry):
        u = wid * _UPW + ui
        d0 = pl.multiple_of((u // _LB) * _B0, _B0)
        l0 = pl.multiple_of((u % _LB) * _BL, _BL)
        in_pend = [
            pltpu.async_copy(
                xq_hbm.at[pl.ds(d0, _B0), pl.ds(4 * h, 4), :,
                          pl.ds(l0, _BL)],
                in_v.at[:, pl.ds(4 * h, 4)], semi[h])
            for h in range(2)
        ]

        pending = [None, None]
        for o in range(8):
            if o % 4 == 0:
                in_pend[o // 4].wait()
            par = o % 2
            if pending[par] is not None:
                pending[par].wait()

            @plsc.parallel_loop(0, _B0, unroll=2)
            def tbody(d, _o=o, _par=par):
                vals = [(j, li,
                         in_v[d, _o, j, pl.ds(li * 16, 16)])
                        for j in range(8) for li in range(_BL // 16)]
                for j, li, v in vals:
                    cv[_par, j, d, pl.ds(li * 16, 16)] = v
            pending[par] = pltpu.async_copy(
                cv.at[par],
                out_hbm.at[pl.ds(8 * o, 8), pl.ds(d0, _B0),
                           pl.ds(l0, _BL)],
                sems[par])

        prod_pend = []
        for h in range(2):
            ks = range(18 * h, 18 * (h + 1))

            @plsc.parallel_loop(0, _B0, unroll=2)
            def pbody(d, _h=h, _ks=ks):
                need = sorted({int(_P[k, i]) for k in _ks
                               for i in range(2)})
                for li in range(_BL // 16):
                    l = li * 16
                    src = {p: in_v[d, p // 8, p % 8, pl.ds(l, 16)]
                           for p in need}
                    for k in _ks:
                        pv[_h, k - 18 * _h, d, pl.ds(l, 16)] = (
                            src[_P[k, 0]] * src[_P[k, 1]])
            prod_pend.append(pltpu.async_copy(
                pv.at[h],
                out_hbm.at[pl.ds(64 + 18 * h, 18), pl.ds(d0, _B0),
                           pl.ds(l0, _BL)],
                semp[h]))
        for par in range(2):
            if pending[par] is not None:
                pending[par].wait()
        for h in range(2):
            prod_pend[h].wait()
        return carry

    lax.fori_loop(0, _UPW, unit_body, 0)


@jax.jit
def kernel(x):
    xq = jnp.transpose(x, (0, 2, 1)).reshape(_D0, 8, 8, _D1)
    ot = _sc_kernel(xq)
    return jnp.transpose(ot, (1, 2, 0))
